# baseline (speedup 1.0000x reference)
import functools

import jax
import jax.numpy as jnp
from jax.experimental import pallas as pl
from jax.experimental.pallas import tpu as pltpu

ALPHA = 0.1
N = 16384
D = 64
BM = 128
NBUF = 4
NSTEPS = N // BM


def _prop_kernel(adj_hbm, inp_ref, h_ref, out_ref, abuf, sems):
    def band_copy(step, slot):
        return pltpu.make_async_copy(
            adj_hbm.at[pl.ds(step * BM, BM), :],
            abuf.at[slot],
            sems.at[slot],
        )

    for s in range(NBUF):
        band_copy(s, s).start()

    def body(i, carry):
        slot = jax.lax.rem(i, NBUF)
        band_copy(i, slot).wait()
        rows = pl.ds(i * BM, BM)
        out_ref[rows, :] = (1.0 - ALPHA) * jnp.dot(
            abuf[slot], inp_ref[...], preferred_element_type=jnp.float32
        ) + ALPHA * h_ref[rows, :]

        @pl.when(i + NBUF < NSTEPS)
        def _refill():
            band_copy(i + NBUF, slot).start()

        return carry

    jax.lax.fori_loop(0, NSTEPS, body, 0)


@functools.partial(jax.jit, static_argnames=())
def kernel(input, adj, h, W):
    del W
    return pl.pallas_call(
        _prop_kernel,
        in_specs=[
            pl.BlockSpec(memory_space=pltpu.MemorySpace.HBM),
            pl.BlockSpec(memory_space=pltpu.MemorySpace.VMEM),
            pl.BlockSpec(memory_space=pltpu.MemorySpace.VMEM),
        ],
        out_specs=pl.BlockSpec(memory_space=pltpu.MemorySpace.VMEM),
        out_shape=jax.ShapeDtypeStruct((N, D), jnp.float32),
        scratch_shapes=[
            pltpu.VMEM((NBUF, BM, N), jnp.float32),
            pltpu.SemaphoreType.DMA((NBUF,)),
        ],
    )(adj, input, h)
